# Initial kernel scaffold; baseline (speedup 1.0000x reference)
import jax
import jax.numpy as jnp
from jax.experimental import pallas as pl


def kernel(x, supp_edges, supp_sten, w1, off1, w2, off2, b1, b2, res_wr, res_wi):
    raise NotImplementedError("write your pallas kernel here")


# speedup vs baseline: 93.3639x; 93.3639x over previous
import functools

import jax
import jax.numpy as jnp
from jax import lax
from jax.experimental import pallas as pl
from jax.experimental.pallas import tpu as pltpu
from jax.experimental.pallas import tpu_sc as plsc


def _prep_w_kernel(w_ref, off_ref, wr_ref, wi_ref):
    w = w_ref[...]
    off = off_ref[...]
    wr_ref[...] = w * jnp.cos(off)
    wi_ref[...] = w * jnp.sin(off)


def _prep_w(w_packed, off_packed):
    shape = jax.ShapeDtypeStruct(w_packed.shape, jnp.float32)
    return pl.pallas_call(
        _prep_w_kernel,
        out_shape=(shape, shape),
    )(w_packed, off_packed)


def _mm_v_kernel(xr_ref, xi_ref, wr_ref, wi_ref, out_ref, *, half_cols):
    xr = xr_ref[...]
    xi = xi_ref[...]
    wr = wr_ref[0]
    wi = wi_ref[0]
    vr = (jnp.dot(xr, wr, preferred_element_type=jnp.float32)
          - jnp.dot(xi, wi, preferred_element_type=jnp.float32))
    vi = (jnp.dot(xr, wi, preferred_element_type=jnp.float32)
          + jnp.dot(xi, wr, preferred_element_type=jnp.float32))
    out_ref[0, :, 0:half_cols] = vr
    out_ref[0, :, half_cols:2 * half_cols] = vi


def _mm_v2_kernel(hr_ref, hi_ref, b_ref, wr_ref, wi_ref, out_ref, *, half_cols):
    hr = hr_ref[...]
    hi = hi_ref[...]
    b = b_ref[...]
    mag = jnp.sqrt(hr * hr + hi * hi)
    scale = jax.nn.relu(mag + b) / (mag + 1e-12)
    xr = scale * hr
    xi = scale * hi
    wr = wr_ref[0]
    wi = wi_ref[0]
    vr = (jnp.dot(xr, wr, preferred_element_type=jnp.float32)
          - jnp.dot(xi, wi, preferred_element_type=jnp.float32))
    vi = (jnp.dot(xr, wi, preferred_element_type=jnp.float32)
          + jnp.dot(xi, wr, preferred_element_type=jnp.float32))
    out_ref[0, :, 0:half_cols] = vr
    out_ref[0, :, half_cols:2 * half_cols] = vi


def _mm_v(xr, xi, wr_p, wi_p, bn, fused_b=None):
    n, cin = xr.shape
    half_cols = wr_p.shape[2]
    grid = (2, n // bn)
    x_spec = pl.BlockSpec((bn, cin), lambda h, i: (i, 0))
    w_spec = pl.BlockSpec((1, cin, half_cols), lambda h, i: (h, 0, 0))
    out_spec = pl.BlockSpec((1, bn, 2 * half_cols), lambda h, i: (h, i, 0))
    out_shape = jax.ShapeDtypeStruct((2, n, 2 * half_cols), jnp.float32)
    if fused_b is None:
        fn = functools.partial(_mm_v_kernel, half_cols=half_cols)
        in_specs = [x_spec, x_spec, w_spec, w_spec]
        args = (xr, xi, wr_p, wi_p)
    else:
        fn = functools.partial(_mm_v2_kernel, half_cols=half_cols)
        b_spec = pl.BlockSpec((1, cin), lambda h, i: (0, 0))
        in_specs = [x_spec, x_spec, b_spec, w_spec, w_spec]
        args = (xr, xi, fused_b, wr_p, wi_p)
    out = pl.pallas_call(
        fn,
        grid=grid,
        in_specs=in_specs,
        out_specs=out_spec,
        out_shape=out_shape,
    )(*args)
    return out.reshape(2 * n, 2 * half_cols)


def _final_kernel(xr_ref, xi_ref, rwr_ref, rwi_ref, hr_ref, hi_ref, b_ref,
                  or_ref, oi_ref):
    xr = xr_ref[...]
    xi = xi_ref[...]
    rwr = rwr_ref[...]
    rwi = rwi_ref[...]
    rr = (jnp.dot(xr, rwr, preferred_element_type=jnp.float32)
          - jnp.dot(xi, rwi, preferred_element_type=jnp.float32)) + hr_ref[...]
    ri = (jnp.dot(xr, rwi, preferred_element_type=jnp.float32)
          + jnp.dot(xi, rwr, preferred_element_type=jnp.float32)) + hi_ref[...]
    b = b_ref[...]
    mag = jnp.sqrt(rr * rr + ri * ri)
    scale = jax.nn.relu(mag + b) / (mag + 1e-12)
    or_ref[...] = scale * rr
    oi_ref[...] = scale * ri


def _final(xr, xi, rwr, rwi, hr, hi, b2, bn):
    n, cin = xr.shape
    cout = rwr.shape[1]
    grid = (n // bn,)
    x_spec = pl.BlockSpec((bn, cin), lambda i: (i, 0))
    w_spec = pl.BlockSpec((cin, cout), lambda i: (0, 0))
    h_spec = pl.BlockSpec((bn, cout), lambda i: (i, 0))
    b_spec = pl.BlockSpec((1, cout), lambda i: (0, 0))
    out_shape = jax.ShapeDtypeStruct((n, cout), jnp.float32)
    return pl.pallas_call(
        _final_kernel,
        grid=grid,
        in_specs=[x_spec, x_spec, w_spec, w_spec, h_spec, h_spec, b_spec],
        out_specs=(h_spec, h_spec),
        out_shape=(out_shape, out_shape),
    )(xr, xi, rwr, rwi, hr, hi, b2)


_BATCH = 16


def _sc_edge_body(table_hbm, src_hbm, dst_hbm, s_hbm, zeros_hbm, out_hbm,
                  idx_v, dst_v, s_v, rows_v, ob_v, acc_sh, sem,
                  *, n_nodes, e_edges, k_taps, half):
    c = lax.axis_index("c")
    sid = lax.axis_index("s")
    n_sub = 16
    edges_per_tile = e_edges // n_sub
    n_batches = edges_per_tile // _BATCH

    @pl.when(sid == 0)
    def _():
        pltpu.sync_copy(zeros_hbm, acc_sh)

    plsc.subcore_barrier()

    base0 = sid * edges_per_tile
    zero16 = jnp.zeros((16,), jnp.float32)
    zero16i = jnp.zeros((16,), jnp.int32)

    def batch_body(b, carry):
        base = base0 + b * _BATCH
        pltpu.sync_copy(src_hbm.at[pl.ds(base, _BATCH)], idx_v)
        pltpu.sync_copy(dst_hbm.at[pl.ds(base, _BATCH)], dst_v)
        pltpu.sync_copy(s_hbm.at[pl.ds(base, _BATCH)], s_v)
        iv = idx_v[...]
        idx_v[...] = iv + c * n_nodes
        pltpu.async_copy(table_hbm.at[idx_v], rows_v, sem).wait()

        def edge_body(e, carry2):
            srow = [s_v[e, pl.ds(16 * t, 16)] for t in range(3)]

            def bcast(pos):
                dnums = lax.GatherDimensionNumbers(
                    offset_dims=(), collapsed_slice_dims=(0,),
                    start_index_map=(0,))
                return lax.gather(
                    srow[pos // 16], (zero16i + (pos % 16))[:, None], dnums,
                    (1,), mode=lax.GatherScatterMode.PROMISE_IN_BOUNDS)

            acc = [zero16] * 8
            for k in range(k_taps):
                sr = bcast(k)
                si = bcast(k_taps + k)
                for j in range(half // 16):
                    vr = rows_v[e, pl.ds(k * half + j * 16, 16)]
                    vi = rows_v[e, pl.ds(k_taps * half + k * half + j * 16, 16)]
                    acc[j] = acc[j] + sr * vr - si * vi
                    acc[4 + j] = acc[4 + j] + sr * vi + si * vr
            for j in range(half // 16):
                ob_v[e, pl.ds(j * 16, 16)] = acc[j]
                ob_v[e, pl.ds(half + j * 16, 16)] = acc[4 + j]
            return carry2

        lax.fori_loop(0, _BATCH, edge_body, 0, unroll=False)
        pltpu.sync_copy(ob_v, acc_sh.at[dst_v], add=True)
        return carry

    lax.fori_loop(0, n_batches, batch_body, 0, unroll=False)
    plsc.subcore_barrier()

    @pl.when(sid == 0)
    def _():
        pltpu.sync_copy(acc_sh, out_hbm.at[c])


def _sc_edge(table, src, dst, s_flat, zeros_nc, n_nodes, e_edges, k_taps, half):
    mesh = plsc.VectorSubcoreMesh(core_axis_name="c", subcore_axis_name="s")
    row_w = 2 * k_taps * half
    kern = functools.partial(
        pl.kernel,
        mesh=mesh,
        out_type=jax.ShapeDtypeStruct((2, n_nodes, 2 * half), jnp.float32),
        scratch_types=[
            pltpu.VMEM((_BATCH,), jnp.int32),
            pltpu.VMEM((_BATCH,), jnp.int32),
            pltpu.VMEM((_BATCH, 48), jnp.float32),
            pltpu.VMEM((_BATCH, row_w), jnp.float32),
            pltpu.VMEM((_BATCH, 2 * half), jnp.float32),
            pltpu.VMEM_SHARED((n_nodes, 2 * half), jnp.float32),
            pltpu.SemaphoreType.DMA,
        ],
    )
    body = functools.partial(_sc_edge_body, n_nodes=n_nodes, e_edges=e_edges,
                             k_taps=k_taps, half=half)
    return kern(body)(table, src, dst, s_flat, zeros_nc)


def kernel(x, supp_edges, supp_sten, w1, off1, w2, off2, b1, b2, res_wr, res_wi):
    n, cin = x.shape
    e = supp_edges.shape[0]
    k_taps = supp_sten.shape[1] * supp_sten.shape[2]
    cout = w1.shape[2]
    half = cout // 2
    bn = 400

    xr = jnp.real(x)
    xi = jnp.imag(x)
    src = supp_edges[:, 0]
    dst = supp_edges[:, 1]
    sten = supp_sten.reshape(e, k_taps)
    s_flat = jnp.concatenate(
        [jnp.real(sten), jnp.imag(sten),
         jnp.zeros((e, 48 - 2 * k_taps), jnp.float32)], axis=1)
    zeros_nc = jnp.zeros((n, cout), jnp.float32)

    def pack_w(w):
        return (w.transpose(1, 0, 2)
                .reshape(cin, k_taps, 2, half)
                .transpose(2, 0, 1, 3)
                .reshape(2, cin, k_taps * half))

    wr1, wi1 = _prep_w(pack_w(w1), pack_w(off1))
    wr2, wi2 = _prep_w(pack_w(w2), pack_w(off2))

    table1 = _mm_v(xr, xi, wr1, wi1, bn)
    o1 = _sc_edge(table1, src, dst, s_flat, zeros_nc, n, e, k_taps, half)
    o1 = o1.reshape(2, n, cout)
    h1r = jnp.concatenate([o1[0, :, :half], o1[1, :, :half]], axis=1)
    h1i = jnp.concatenate([o1[0, :, half:], o1[1, :, half:]], axis=1)

    table2 = _mm_v(h1r, h1i, wr2, wi2, bn, fused_b=b1.reshape(1, cout))
    o2 = _sc_edge(table2, src, dst, s_flat, zeros_nc, n, e, k_taps, half)
    o2 = o2.reshape(2, n, cout)
    h2r = jnp.concatenate([o2[0, :, :half], o2[1, :, :half]], axis=1)
    h2i = jnp.concatenate([o2[0, :, half:], o2[1, :, half:]], axis=1)

    outr, outi = _final(xr, xi, res_wr, res_wi, h2r, h2i, b2.reshape(1, cout), bn)
    return lax.complex(outr, outi)


# speedup vs baseline: 140.1725x; 1.5014x over previous
import functools

import jax
import jax.numpy as jnp
from jax import lax
from jax.experimental import pallas as pl
from jax.experimental.pallas import tpu as pltpu
from jax.experimental.pallas import tpu_sc as plsc


def _prep_w_kernel(w_ref, off_ref, wr_ref, wi_ref):
    w = w_ref[...]
    off = off_ref[...]
    wr_ref[...] = w * jnp.cos(off)
    wi_ref[...] = w * jnp.sin(off)


def _prep_w(w_packed, off_packed):
    shape = jax.ShapeDtypeStruct(w_packed.shape, jnp.float32)
    return pl.pallas_call(
        _prep_w_kernel,
        out_shape=(shape, shape),
    )(w_packed, off_packed)


def _mm_v_kernel(xr_ref, xi_ref, wr_ref, wi_ref, out_ref, *, half_cols):
    xr = xr_ref[...]
    xi = xi_ref[...]
    wr = wr_ref[0, 0]
    wi = wi_ref[0, 0]
    vr = (jnp.dot(xr, wr, preferred_element_type=jnp.float32)
          - jnp.dot(xi, wi, preferred_element_type=jnp.float32))
    vi = (jnp.dot(xr, wi, preferred_element_type=jnp.float32)
          + jnp.dot(xi, wr, preferred_element_type=jnp.float32))
    out_ref[0, 0, :, 0:half_cols] = vr
    out_ref[0, 0, :, half_cols:2 * half_cols] = vi


def _mm_v2_kernel(hr_ref, hi_ref, b_ref, wr_ref, wi_ref, out_ref, *, half_cols):
    hr = hr_ref[...]
    hi = hi_ref[...]
    b = b_ref[...]
    mag = jnp.sqrt(hr * hr + hi * hi)
    scale = jax.nn.relu(mag + b) / (mag + 1e-12)
    xr = scale * hr
    xi = scale * hi
    wr = wr_ref[0, 0]
    wi = wi_ref[0, 0]
    vr = (jnp.dot(xr, wr, preferred_element_type=jnp.float32)
          - jnp.dot(xi, wi, preferred_element_type=jnp.float32))
    vi = (jnp.dot(xr, wi, preferred_element_type=jnp.float32)
          + jnp.dot(xi, wr, preferred_element_type=jnp.float32))
    out_ref[0, 0, :, 0:half_cols] = vr
    out_ref[0, 0, :, half_cols:2 * half_cols] = vi


def _mm_v(xr, xi, wr_p, wi_p, bn, fused_b=None):
    n, cin = xr.shape
    half_cols = wr_p.shape[3]
    grid = (2, 2, n // bn)
    x_spec = pl.BlockSpec((bn, cin), lambda kh, h, i: (i, 0))
    w_spec = pl.BlockSpec((1, 1, cin, half_cols),
                          lambda kh, h, i: (kh, h, 0, 0))
    out_spec = pl.BlockSpec((1, 1, bn, 2 * half_cols),
                            lambda kh, h, i: (kh, h, i, 0))
    out_shape = jax.ShapeDtypeStruct((2, 2, n, 2 * half_cols), jnp.float32)
    if fused_b is None:
        fn = functools.partial(_mm_v_kernel, half_cols=half_cols)
        in_specs = [x_spec, x_spec, w_spec, w_spec]
        args = (xr, xi, wr_p, wi_p)
    else:
        fn = functools.partial(_mm_v2_kernel, half_cols=half_cols)
        b_spec = pl.BlockSpec((1, cin), lambda kh, h, i: (0, 0))
        in_specs = [x_spec, x_spec, b_spec, w_spec, w_spec]
        args = (xr, xi, fused_b, wr_p, wi_p)
    out = pl.pallas_call(
        fn,
        grid=grid,
        in_specs=in_specs,
        out_specs=out_spec,
        out_shape=out_shape,
    )(*args)
    return out.reshape(2, 2 * n, 2 * half_cols)


def _final_kernel(xr_ref, xi_ref, rwr_ref, rwi_ref, hr_ref, hi_ref, b_ref,
                  or_ref, oi_ref):
    xr = xr_ref[...]
    xi = xi_ref[...]
    rwr = rwr_ref[...]
    rwi = rwi_ref[...]
    rr = (jnp.dot(xr, rwr, preferred_element_type=jnp.float32)
          - jnp.dot(xi, rwi, preferred_element_type=jnp.float32)) + hr_ref[...]
    ri = (jnp.dot(xr, rwi, preferred_element_type=jnp.float32)
          + jnp.dot(xi, rwr, preferred_element_type=jnp.float32)) + hi_ref[...]
    b = b_ref[...]
    mag = jnp.sqrt(rr * rr + ri * ri)
    scale = jax.nn.relu(mag + b) / (mag + 1e-12)
    or_ref[...] = scale * rr
    oi_ref[...] = scale * ri


def _final(xr, xi, rwr, rwi, hr, hi, b2, bn):
    n, cin = xr.shape
    cout = rwr.shape[1]
    grid = (n // bn,)
    x_spec = pl.BlockSpec((bn, cin), lambda i: (i, 0))
    w_spec = pl.BlockSpec((cin, cout), lambda i: (0, 0))
    h_spec = pl.BlockSpec((bn, cout), lambda i: (i, 0))
    b_spec = pl.BlockSpec((1, cout), lambda i: (0, 0))
    out_shape = jax.ShapeDtypeStruct((n, cout), jnp.float32)
    return pl.pallas_call(
        _final_kernel,
        grid=grid,
        in_specs=[x_spec, x_spec, w_spec, w_spec, h_spec, h_spec, b_spec],
        out_specs=(h_spec, h_spec),
        out_shape=(out_shape, out_shape),
    )(xr, xi, rwr, rwi, hr, hi, b2)


_BATCH = 16


_CHUNK_B = 10


_QB = 8


def _sc_edge_body(table_hbm, src_hbm, dst_hbm, s_hbm, zeros_hbm, out_hbm,
                  src_c, dstl_c, dst_c, s_c, rows_v, ob_v, acc_sh,
                  gsem0, gsem1, ssem,
                  *, n_nodes, e_edges, k_taps, half):
    c = lax.axis_index("c")
    sid = lax.axis_index("s")
    n_sub = 16
    edges_per_tile = e_edges // n_sub
    chunk_e = _CHUNK_B * _BATCH
    n_chunks = edges_per_tile // chunk_e

    @pl.when(sid == 0)
    def _():
        pltpu.sync_copy(zeros_hbm, acc_sh)

    plsc.subcore_barrier()

    zero16 = jnp.zeros((16,), jnp.float32)
    zero16i = jnp.zeros((16,), jnp.int32)
    gsems = (gsem0, gsem1)
    kh_taps = k_taps // 2

    def compute_half(b_row, h, kh, acc_add):
        def edge_body(e, carry2):
            srow = [s_c[b_row * _BATCH + h * _QB + e, pl.ds(16 * t, 16)]
                    for t in range(3)]

            def bcast(pos):
                dnums = lax.GatherDimensionNumbers(
                    offset_dims=(), collapsed_slice_dims=(0,),
                    start_index_map=(0,))
                return lax.gather(
                    srow[pos // 16], (zero16i + (pos % 16))[:, None], dnums,
                    (1,), mode=lax.GatherScatterMode.PROMISE_IN_BOUNDS)

            ob_row = h * _QB + e
            if acc_add:
                acc = ([ob_v[ob_row, pl.ds(j * 16, 16)]
                        for j in range(half // 16)] +
                       [ob_v[ob_row, pl.ds(half + j * 16, 16)]
                        for j in range(half // 16)])
            else:
                acc = [zero16] * 8
            for k in range(kh_taps):
                sr = bcast(kh * kh_taps + k)
                si = bcast(k_taps + kh * kh_taps + k)
                for j in range(half // 16):
                    vr = rows_v[h, e, pl.ds(k * half + j * 16, 16)]
                    vi = rows_v[h, e,
                                pl.ds(kh_taps * half + k * half + j * 16, 16)]
                    acc[j] = acc[j] + sr * vr - si * vi
                    acc[4 + j] = acc[4 + j] + sr * vi + si * vr
            for j in range(half // 16):
                ob_v[ob_row, pl.ds(j * 16, 16)] = acc[j]
                ob_v[ob_row, pl.ds(half + j * 16, 16)] = acc[4 + j]
            return carry2

        lax.fori_loop(0, _QB, edge_body, 0, unroll=False)

    def load_chunk(ch):
        base = sid * edges_per_tile + ch * chunk_e
        pltpu.sync_copy(src_hbm.at[pl.ds(base, chunk_e)], src_c)
        pltpu.sync_copy(dst_hbm.at[pl.ds(base, chunk_e)], dstl_c)
        pltpu.sync_copy(s_hbm.at[pl.ds(base, chunk_e)], s_c)
        for t in range(_CHUNK_B):
            src_c[pl.ds(t * _BATCH, _BATCH)] = (
                src_c[pl.ds(t * _BATCH, _BATCH)] + c * n_nodes)
            dst_c[t] = dstl_c[pl.ds(t * _BATCH, _BATCH)]

    def fire_gather(b_row, s):
        h, kh = s % 2, s // 2
        pltpu.async_copy(
            table_hbm.at[kh].at[
                src_c.at[pl.ds(b_row * _BATCH + h * _QB, _QB)]],
            rows_v.at[h], gsems[h])

    def wait_gather(b_row, s):
        h, kh = s % 2, s // 2
        pltpu.make_async_copy(
            table_hbm.at[kh].at[
                src_c.at[pl.ds(b_row * _BATCH + h * _QB, _QB)]],
            rows_v.at[h], gsems[h]).wait()

    def drain_scatter():
        pltpu.make_async_copy(ob_v, acc_sh.at[dst_c.at[0]], ssem).wait()

    def chunk_body(ch, carry):
        load_chunk(ch)
        fire_gather(0, 0)

        def batch_body(b, carry2):
            for s in range(4):
                if s + 1 < 4:
                    fire_gather(b, s + 1)
                else:
                    @pl.when(b < _CHUNK_B - 1)
                    def _():
                        fire_gather(b + 1, 0)

                wait_gather(b, s)
                if s == 0:
                    @pl.when(b > 0)
                    def _():
                        drain_scatter()

                compute_half(b, s % 2, s // 2, acc_add=(s // 2 == 1))
            pltpu.async_copy(ob_v, acc_sh.at[dst_c.at[b]], ssem, add=True)
            return carry2

        lax.fori_loop(0, _CHUNK_B, batch_body, 0, unroll=False)
        drain_scatter()
        return carry

    lax.fori_loop(0, n_chunks, chunk_body, 0, unroll=False)
    plsc.subcore_barrier()

    @pl.when(sid == 0)
    def _():
        pltpu.sync_copy(acc_sh, out_hbm.at[c])


def _sc_edge(table, src, dst, s_flat, zeros_nc, n_nodes, e_edges, k_taps, half):
    mesh = plsc.VectorSubcoreMesh(core_axis_name="c", subcore_axis_name="s")
    row_w = k_taps * half
    kern = functools.partial(
        pl.kernel,
        mesh=mesh,
        out_type=jax.ShapeDtypeStruct((2, n_nodes, 2 * half), jnp.float32),
        scratch_types=[
            pltpu.VMEM((_CHUNK_B * _BATCH,), jnp.int32),
            pltpu.VMEM((_CHUNK_B * _BATCH,), jnp.int32),
            pltpu.VMEM((_CHUNK_B, _BATCH), jnp.int32),
            pltpu.VMEM((_CHUNK_B * _BATCH, 48), jnp.float32),
            pltpu.VMEM((2, _QB, row_w), jnp.float32),
            pltpu.VMEM((_BATCH, 2 * half), jnp.float32),
            pltpu.VMEM_SHARED((n_nodes, 2 * half), jnp.float32),
            pltpu.SemaphoreType.DMA,
            pltpu.SemaphoreType.DMA,
            pltpu.SemaphoreType.DMA,
        ],
    )
    body = functools.partial(_sc_edge_body, n_nodes=n_nodes, e_edges=e_edges,
                             k_taps=k_taps, half=half)
    return kern(body)(table, src, dst, s_flat, zeros_nc)


def kernel(x, supp_edges, supp_sten, w1, off1, w2, off2, b1, b2, res_wr, res_wi):
    n, cin = x.shape
    e = supp_edges.shape[0]
    k_taps = supp_sten.shape[1] * supp_sten.shape[2]
    cout = w1.shape[2]
    half = cout // 2
    bn = 400

    xr = jnp.real(x)
    xi = jnp.imag(x)
    src = supp_edges[:, 0]
    dst = supp_edges[:, 1]
    sten = supp_sten.reshape(e, k_taps)
    s_flat = jnp.concatenate(
        [jnp.real(sten), jnp.imag(sten),
         jnp.zeros((e, 48 - 2 * k_taps), jnp.float32)], axis=1)
    zeros_nc = jnp.zeros((n, cout), jnp.float32)

    kh_taps = k_taps // 2

    def pack_w(w):
        return (w.transpose(1, 0, 2)
                .reshape(cin, 2, kh_taps, 2, half)
                .transpose(1, 3, 0, 2, 4)
                .reshape(2, 2, cin, kh_taps * half))

    wr1, wi1 = _prep_w(pack_w(w1), pack_w(off1))
    wr2, wi2 = _prep_w(pack_w(w2), pack_w(off2))

    table1 = _mm_v(xr, xi, wr1, wi1, bn)
    o1 = _sc_edge(table1, src, dst, s_flat, zeros_nc, n, e, k_taps, half)
    o1 = o1.reshape(2, n, cout)
    h1r = jnp.concatenate([o1[0, :, :half], o1[1, :, :half]], axis=1)
    h1i = jnp.concatenate([o1[0, :, half:], o1[1, :, half:]], axis=1)

    table2 = _mm_v(h1r, h1i, wr2, wi2, bn, fused_b=b1.reshape(1, cout))
    o2 = _sc_edge(table2, src, dst, s_flat, zeros_nc, n, e, k_taps, half)
    o2 = o2.reshape(2, n, cout)
    h2r = jnp.concatenate([o2[0, :, :half], o2[1, :, :half]], axis=1)
    h2i = jnp.concatenate([o2[0, :, half:], o2[1, :, half:]], axis=1)

    outr, outi = _final(xr, xi, res_wr, res_wi, h2r, h2i, b2.reshape(1, cout), bn)
    return lax.complex(outr, outi)


# speedup vs baseline: 169.0599x; 1.2061x over previous
import functools

import jax
import jax.numpy as jnp
from jax import lax
from jax.experimental import pallas as pl
from jax.experimental.pallas import tpu as pltpu
from jax.experimental.pallas import tpu_sc as plsc


def _prep_w_kernel(w_ref, off_ref, wr_ref, wi_ref):
    w = w_ref[...]
    off = off_ref[...]
    wr_ref[...] = w * jnp.cos(off)
    wi_ref[...] = w * jnp.sin(off)


def _prep_w(w_packed, off_packed):
    shape = jax.ShapeDtypeStruct(w_packed.shape, jnp.float32)
    return pl.pallas_call(
        _prep_w_kernel,
        out_shape=(shape, shape),
    )(w_packed, off_packed)


def _mm_v_kernel(xr_ref, xi_ref, wr_ref, wi_ref, out_ref, *, half_cols):
    xr = xr_ref[...]
    xi = xi_ref[...]
    wr = wr_ref[0]
    wi = wi_ref[0]
    vr = (jnp.dot(xr, wr, preferred_element_type=jnp.float32)
          - jnp.dot(xi, wi, preferred_element_type=jnp.float32))
    vi = (jnp.dot(xr, wi, preferred_element_type=jnp.float32)
          + jnp.dot(xi, wr, preferred_element_type=jnp.float32))
    out_ref[0] = _pack_ri(vr, vi)


def _pack_ri(vr, vi):
    hi = lax.bitcast_convert_type(vr.astype(jnp.bfloat16),
                                  jnp.uint16).astype(jnp.uint32)
    lo = lax.bitcast_convert_type(vi.astype(jnp.bfloat16),
                                  jnp.uint16).astype(jnp.uint32)
    return lax.bitcast_convert_type(
        jnp.bitwise_or(lax.shift_left(hi, jnp.uint32(16)), lo), jnp.int32)


def _mm_v2_kernel(hr_ref, hi_ref, b_ref, wr_ref, wi_ref, out_ref, *, half_cols):
    hr = hr_ref[...]
    hi = hi_ref[...]
    b = b_ref[...]
    mag = jnp.sqrt(hr * hr + hi * hi)
    scale = jax.nn.relu(mag + b) / (mag + 1e-12)
    xr = scale * hr
    xi = scale * hi
    wr = wr_ref[0]
    wi = wi_ref[0]
    vr = (jnp.dot(xr, wr, preferred_element_type=jnp.float32)
          - jnp.dot(xi, wi, preferred_element_type=jnp.float32))
    vi = (jnp.dot(xr, wi, preferred_element_type=jnp.float32)
          + jnp.dot(xi, wr, preferred_element_type=jnp.float32))
    out_ref[0] = _pack_ri(vr, vi)


def _mm_v(xr, xi, wr_p, wi_p, bn, fused_b=None):
    n, cin = xr.shape
    cols = wr_p.shape[2]
    grid = (2, n // bn)
    x_spec = pl.BlockSpec((bn, cin), lambda h, i: (i, 0))
    w_spec = pl.BlockSpec((1, cin, cols), lambda h, i: (h, 0, 0))
    out_spec = pl.BlockSpec((1, bn, cols), lambda h, i: (h, i, 0))
    out_shape = jax.ShapeDtypeStruct((2, n, cols), jnp.int32)
    if fused_b is None:
        fn = functools.partial(_mm_v_kernel, half_cols=cols)
        in_specs = [x_spec, x_spec, w_spec, w_spec]
        args = (xr, xi, wr_p, wi_p)
    else:
        fn = functools.partial(_mm_v2_kernel, half_cols=cols)
        b_spec = pl.BlockSpec((1, cin), lambda h, i: (0, 0))
        in_specs = [x_spec, x_spec, b_spec, w_spec, w_spec]
        args = (xr, xi, fused_b, wr_p, wi_p)
    out = pl.pallas_call(
        fn,
        grid=grid,
        in_specs=in_specs,
        out_specs=out_spec,
        out_shape=out_shape,
    )(*args)
    return out.reshape(2 * n, cols)


def _final_kernel(xr_ref, xi_ref, rwr_ref, rwi_ref, hr_ref, hi_ref, b_ref,
                  or_ref, oi_ref):
    xr = xr_ref[...]
    xi = xi_ref[...]
    rwr = rwr_ref[...]
    rwi = rwi_ref[...]
    rr = (jnp.dot(xr, rwr, preferred_element_type=jnp.float32)
          - jnp.dot(xi, rwi, preferred_element_type=jnp.float32)) + hr_ref[...]
    ri = (jnp.dot(xr, rwi, preferred_element_type=jnp.float32)
          + jnp.dot(xi, rwr, preferred_element_type=jnp.float32)) + hi_ref[...]
    b = b_ref[...]
    mag = jnp.sqrt(rr * rr + ri * ri)
    scale = jax.nn.relu(mag + b) / (mag + 1e-12)
    or_ref[...] = scale * rr
    oi_ref[...] = scale * ri


def _final(xr, xi, rwr, rwi, hr, hi, b2, bn):
    n, cin = xr.shape
    cout = rwr.shape[1]
    grid = (n // bn,)
    x_spec = pl.BlockSpec((bn, cin), lambda i: (i, 0))
    w_spec = pl.BlockSpec((cin, cout), lambda i: (0, 0))
    h_spec = pl.BlockSpec((bn, cout), lambda i: (i, 0))
    b_spec = pl.BlockSpec((1, cout), lambda i: (0, 0))
    out_shape = jax.ShapeDtypeStruct((n, cout), jnp.float32)
    return pl.pallas_call(
        _final_kernel,
        grid=grid,
        in_specs=[x_spec, x_spec, w_spec, w_spec, h_spec, h_spec, b_spec],
        out_specs=(h_spec, h_spec),
        out_shape=(out_shape, out_shape),
    )(xr, xi, rwr, rwi, hr, hi, b2)


_BATCH = 16


_CHUNK_B = 10


_QB = 8


def _sc_edge_body(table_hbm, src_hbm, dst_hbm, s_hbm, zeros_hbm, out_hbm,
                  src_c, dstl_c, dst_c, s_c, rows_v, ob_v, acc_sh,
                  gsem0, gsem1, ssem,
                  *, n_nodes, e_edges, k_taps, half):
    c = lax.axis_index("c")
    sid = lax.axis_index("s")
    n_sub = 16
    edges_per_tile = e_edges // n_sub
    chunk_e = _CHUNK_B * _BATCH
    n_chunks = edges_per_tile // chunk_e

    @pl.when(sid == 0)
    def _():
        pltpu.sync_copy(zeros_hbm, acc_sh)

    plsc.subcore_barrier()

    zero16 = jnp.zeros((16,), jnp.float32)
    zero16i = jnp.zeros((16,), jnp.int32)
    gsems = (gsem0, gsem1)
    mask_hi = jnp.full((16,), -65536, jnp.int32)

    def compute_half(b_row, h):
        def edge_body(e, carry2):
            srow = [s_c[b_row * _BATCH + h * _QB + e, pl.ds(16 * t, 16)]
                    for t in range(3)]

            def bcast(pos):
                dnums = lax.GatherDimensionNumbers(
                    offset_dims=(), collapsed_slice_dims=(0,),
                    start_index_map=(0,))
                return lax.gather(
                    srow[pos // 16], (zero16i + (pos % 16))[:, None], dnums,
                    (1,), mode=lax.GatherScatterMode.PROMISE_IN_BOUNDS)

            ob_row = h * _QB + e
            acc = [zero16] * 8
            for k in range(k_taps):
                sr = bcast(k)
                si = bcast(k_taps + k)
                for j in range(half // 16):
                    w = rows_v[h, e, pl.ds(k * half + j * 16, 16)]
                    vr = lax.bitcast_convert_type(
                        jnp.bitwise_and(w, mask_hi), jnp.float32)
                    vi = lax.bitcast_convert_type(
                        lax.shift_left(w, zero16i + 16), jnp.float32)
                    acc[j] = acc[j] + sr * vr - si * vi
                    acc[4 + j] = acc[4 + j] + sr * vi + si * vr
            for j in range(half // 16):
                ob_v[ob_row, pl.ds(j * 16, 16)] = acc[j]
                ob_v[ob_row, pl.ds(half + j * 16, 16)] = acc[4 + j]
            return carry2

        lax.fori_loop(0, _QB, edge_body, 0, unroll=False)

    def load_chunk(ch):
        base = sid * edges_per_tile + ch * chunk_e
        pltpu.sync_copy(src_hbm.at[pl.ds(base, chunk_e)], src_c)
        pltpu.sync_copy(dst_hbm.at[pl.ds(base, chunk_e)], dstl_c)
        pltpu.sync_copy(s_hbm.at[pl.ds(base, chunk_e)], s_c)
        for t in range(_CHUNK_B):
            src_c[pl.ds(t * _BATCH, _BATCH)] = (
                src_c[pl.ds(t * _BATCH, _BATCH)] + c * n_nodes)
            dst_c[t] = dstl_c[pl.ds(t * _BATCH, _BATCH)]

    def fire_gather(b_row, h):
        pltpu.async_copy(
            table_hbm.at[src_c.at[pl.ds(b_row * _BATCH + h * _QB, _QB)]],
            rows_v.at[h], gsems[h])

    def wait_gather(b_row, h):
        pltpu.make_async_copy(
            table_hbm.at[src_c.at[pl.ds(b_row * _BATCH + h * _QB, _QB)]],
            rows_v.at[h], gsems[h]).wait()

    def drain_scatter():
        pltpu.make_async_copy(ob_v, acc_sh.at[dst_c.at[0]], ssem).wait()

    def chunk_body(ch, carry):
        load_chunk(ch)
        fire_gather(0, 0)

        def batch_body(b, carry2):
            fire_gather(b, 1)
            wait_gather(b, 0)

            @pl.when(b > 0)
            def _():
                drain_scatter()

            compute_half(b, 0)

            @pl.when(b < _CHUNK_B - 1)
            def _():
                fire_gather(b + 1, 0)

            wait_gather(b, 1)
            compute_half(b, 1)
            pltpu.async_copy(ob_v, acc_sh.at[dst_c.at[b]], ssem, add=True)
            return carry2

        lax.fori_loop(0, _CHUNK_B, batch_body, 0, unroll=False)
        drain_scatter()
        return carry

    lax.fori_loop(0, n_chunks, chunk_body, 0, unroll=False)
    plsc.subcore_barrier()

    @pl.when(sid == 0)
    def _():
        pltpu.sync_copy(acc_sh, out_hbm.at[c])


def _sc_edge(table, src, dst, s_flat, zeros_nc, n_nodes, e_edges, k_taps, half):
    mesh = plsc.VectorSubcoreMesh(core_axis_name="c", subcore_axis_name="s")
    row_w = k_taps * half
    kern = functools.partial(
        pl.kernel,
        mesh=mesh,
        out_type=jax.ShapeDtypeStruct((2, n_nodes, 2 * half), jnp.float32),
        scratch_types=[
            pltpu.VMEM((_CHUNK_B * _BATCH,), jnp.int32),
            pltpu.VMEM((_CHUNK_B * _BATCH,), jnp.int32),
            pltpu.VMEM((_CHUNK_B, _BATCH), jnp.int32),
            pltpu.VMEM((_CHUNK_B * _BATCH, 48), jnp.float32),
            pltpu.VMEM((2, _QB, row_w), jnp.int32),
            pltpu.VMEM((_BATCH, 2 * half), jnp.float32),
            pltpu.VMEM_SHARED((n_nodes, 2 * half), jnp.float32),
            pltpu.SemaphoreType.DMA,
            pltpu.SemaphoreType.DMA,
            pltpu.SemaphoreType.DMA,
        ],
    )
    body = functools.partial(_sc_edge_body, n_nodes=n_nodes, e_edges=e_edges,
                             k_taps=k_taps, half=half)
    return kern(body)(table, src, dst, s_flat, zeros_nc)


def kernel(x, supp_edges, supp_sten, w1, off1, w2, off2, b1, b2, res_wr, res_wi):
    n, cin = x.shape
    e = supp_edges.shape[0]
    k_taps = supp_sten.shape[1] * supp_sten.shape[2]
    cout = w1.shape[2]
    half = cout // 2
    bn = 400

    xr = jnp.real(x)
    xi = jnp.imag(x)
    src = supp_edges[:, 0]
    dst = supp_edges[:, 1]
    sten = supp_sten.reshape(e, k_taps)
    s_flat = jnp.concatenate(
        [jnp.real(sten), jnp.imag(sten),
         jnp.zeros((e, 48 - 2 * k_taps), jnp.float32)], axis=1)
    zeros_nc = jnp.zeros((n, cout), jnp.float32)

    def pack_w(w):
        return (w.transpose(1, 0, 2)
                .reshape(cin, k_taps, 2, half)
                .transpose(2, 0, 1, 3)
                .reshape(2, cin, k_taps * half))

    wr1, wi1 = _prep_w(pack_w(w1), pack_w(off1))
    wr2, wi2 = _prep_w(pack_w(w2), pack_w(off2))

    table1 = _mm_v(xr, xi, wr1, wi1, bn)
    o1 = _sc_edge(table1, src, dst, s_flat, zeros_nc, n, e, k_taps, half)
    o1 = o1.reshape(2, n, cout)
    h1r = jnp.concatenate([o1[0, :, :half], o1[1, :, :half]], axis=1)
    h1i = jnp.concatenate([o1[0, :, half:], o1[1, :, half:]], axis=1)

    table2 = _mm_v(h1r, h1i, wr2, wi2, bn, fused_b=b1.reshape(1, cout))
    o2 = _sc_edge(table2, src, dst, s_flat, zeros_nc, n, e, k_taps, half)
    o2 = o2.reshape(2, n, cout)
    h2r = jnp.concatenate([o2[0, :, :half], o2[1, :, :half]], axis=1)
    h2i = jnp.concatenate([o2[0, :, half:], o2[1, :, half:]], axis=1)

    outr, outi = _final(xr, xi, res_wr, res_wi, h2r, h2i, b2.reshape(1, cout), bn)
    return lax.complex(outr, outi)


# speedup vs baseline: 169.8216x; 1.0045x over previous
import functools

import jax
import jax.numpy as jnp
from jax import lax
from jax.experimental import pallas as pl
from jax.experimental.pallas import tpu as pltpu
from jax.experimental.pallas import tpu_sc as plsc


def _prep_w_kernel(w_ref, off_ref, wr_ref, wi_ref):
    w = w_ref[...]
    off = off_ref[...]
    wr_ref[...] = w * jnp.cos(off)
    wi_ref[...] = w * jnp.sin(off)


def _prep_w(w_packed, off_packed):
    shape = jax.ShapeDtypeStruct(w_packed.shape, jnp.float32)
    return pl.pallas_call(
        _prep_w_kernel,
        out_shape=(shape, shape),
    )(w_packed, off_packed)


def _mm_v_kernel(xr_ref, xi_ref, wr_ref, wi_ref, out_ref, *, half_cols):
    xr = xr_ref[...]
    xi = xi_ref[...]
    wr = wr_ref[0]
    wi = wi_ref[0]
    vr = (jnp.dot(xr, wr, preferred_element_type=jnp.float32)
          - jnp.dot(xi, wi, preferred_element_type=jnp.float32))
    vi = (jnp.dot(xr, wi, preferred_element_type=jnp.float32)
          + jnp.dot(xi, wr, preferred_element_type=jnp.float32))
    out_ref[0] = _pack_ri(vr, vi)


def _pack_ri(vr, vi):
    hi = lax.bitcast_convert_type(vr.astype(jnp.bfloat16),
                                  jnp.uint16).astype(jnp.uint32)
    lo = lax.bitcast_convert_type(vi.astype(jnp.bfloat16),
                                  jnp.uint16).astype(jnp.uint32)
    return lax.bitcast_convert_type(
        jnp.bitwise_or(lax.shift_left(hi, jnp.uint32(16)), lo), jnp.int32)


def _mm_v2_kernel(hr_ref, hi_ref, b_ref, wr_ref, wi_ref, out_ref, *, half_cols):
    hr = hr_ref[...]
    hi = hi_ref[...]
    b = b_ref[...]
    mag = jnp.sqrt(hr * hr + hi * hi)
    scale = jax.nn.relu(mag + b) / (mag + 1e-12)
    xr = scale * hr
    xi = scale * hi
    wr = wr_ref[0]
    wi = wi_ref[0]
    vr = (jnp.dot(xr, wr, preferred_element_type=jnp.float32)
          - jnp.dot(xi, wi, preferred_element_type=jnp.float32))
    vi = (jnp.dot(xr, wi, preferred_element_type=jnp.float32)
          + jnp.dot(xi, wr, preferred_element_type=jnp.float32))
    out_ref[0] = _pack_ri(vr, vi)


def _mm_v(xr, xi, wr_p, wi_p, bn, fused_b=None):
    n, cin = xr.shape
    cols = wr_p.shape[2]
    grid = (2, n // bn)
    x_spec = pl.BlockSpec((bn, cin), lambda h, i: (i, 0))
    w_spec = pl.BlockSpec((1, cin, cols), lambda h, i: (h, 0, 0))
    out_spec = pl.BlockSpec((1, bn, cols), lambda h, i: (h, i, 0))
    out_shape = jax.ShapeDtypeStruct((2, n, cols), jnp.int32)
    if fused_b is None:
        fn = functools.partial(_mm_v_kernel, half_cols=cols)
        in_specs = [x_spec, x_spec, w_spec, w_spec]
        args = (xr, xi, wr_p, wi_p)
    else:
        fn = functools.partial(_mm_v2_kernel, half_cols=cols)
        b_spec = pl.BlockSpec((1, cin), lambda h, i: (0, 0))
        in_specs = [x_spec, x_spec, b_spec, w_spec, w_spec]
        args = (xr, xi, fused_b, wr_p, wi_p)
    out = pl.pallas_call(
        fn,
        grid=grid,
        in_specs=in_specs,
        out_specs=out_spec,
        out_shape=out_shape,
    )(*args)
    return out.reshape(2 * n, cols)


def _final_kernel(xr_ref, xi_ref, rwr_ref, rwi_ref, hr_ref, hi_ref, b_ref,
                  or_ref, oi_ref):
    xr = xr_ref[...]
    xi = xi_ref[...]
    rwr = rwr_ref[...]
    rwi = rwi_ref[...]
    rr = (jnp.dot(xr, rwr, preferred_element_type=jnp.float32)
          - jnp.dot(xi, rwi, preferred_element_type=jnp.float32)) + hr_ref[...]
    ri = (jnp.dot(xr, rwi, preferred_element_type=jnp.float32)
          + jnp.dot(xi, rwr, preferred_element_type=jnp.float32)) + hi_ref[...]
    b = b_ref[...]
    mag = jnp.sqrt(rr * rr + ri * ri)
    scale = jax.nn.relu(mag + b) / (mag + 1e-12)
    or_ref[...] = scale * rr
    oi_ref[...] = scale * ri


def _final(xr, xi, rwr, rwi, hr, hi, b2, bn):
    n, cin = xr.shape
    cout = rwr.shape[1]
    grid = (n // bn,)
    x_spec = pl.BlockSpec((bn, cin), lambda i: (i, 0))
    w_spec = pl.BlockSpec((cin, cout), lambda i: (0, 0))
    h_spec = pl.BlockSpec((bn, cout), lambda i: (i, 0))
    b_spec = pl.BlockSpec((1, cout), lambda i: (0, 0))
    out_shape = jax.ShapeDtypeStruct((n, cout), jnp.float32)
    return pl.pallas_call(
        _final_kernel,
        grid=grid,
        in_specs=[x_spec, x_spec, w_spec, w_spec, h_spec, h_spec, b_spec],
        out_specs=(h_spec, h_spec),
        out_shape=(out_shape, out_shape),
    )(xr, xi, rwr, rwi, hr, hi, b2)


_BATCH = 16


_CHUNK_B = 10


_QB = 8


def _sc_edge_body(table_hbm, src_hbm, dst_hbm, s_hbm, zeros_hbm, out_hbm,
                  src_c, dstl_c, dst_c, s_c, rows_v, ob_v, acc_sh,
                  gsem0, gsem1, ssem,
                  *, n_nodes, e_edges, k_taps, half):
    c = lax.axis_index("c")
    sid = lax.axis_index("s")
    n_sub = 16
    edges_per_tile = e_edges // n_sub
    chunk_e = _CHUNK_B * _BATCH
    n_chunks = edges_per_tile // chunk_e

    @pl.when(sid == 0)
    def _():
        pltpu.sync_copy(zeros_hbm, acc_sh)

    plsc.subcore_barrier()

    zero16 = jnp.zeros((16,), jnp.float32)
    zero16i = jnp.zeros((16,), jnp.int32)
    gsems = (gsem0, gsem1)
    mask_hi = jnp.full((16,), -65536, jnp.int32)

    def compute_half(b_row, h):
        def edge_body(e, carry2):
            srow = [s_c[b_row * _BATCH + h * _QB + e, pl.ds(16 * t, 16)]
                    for t in range(3)]

            def bcast(pos):
                dnums = lax.GatherDimensionNumbers(
                    offset_dims=(), collapsed_slice_dims=(0,),
                    start_index_map=(0,))
                return lax.gather(
                    srow[pos // 16], (zero16i + (pos % 16))[:, None], dnums,
                    (1,), mode=lax.GatherScatterMode.PROMISE_IN_BOUNDS)

            ob_row = h * _QB + e
            acc = [zero16] * 8
            for k in range(k_taps):
                sr = bcast(k)
                si = bcast(k_taps + k)
                for j in range(half // 16):
                    w = rows_v[h, e, pl.ds(k * half + j * 16, 16)]
                    vr = lax.bitcast_convert_type(
                        jnp.bitwise_and(w, mask_hi), jnp.float32)
                    vi = lax.bitcast_convert_type(
                        lax.shift_left(w, zero16i + 16), jnp.float32)
                    acc[j] = acc[j] + sr * vr - si * vi
                    acc[4 + j] = acc[4 + j] + sr * vi + si * vr
            for j in range(half // 16):
                ob_v[ob_row, pl.ds(j * 16, 16)] = acc[j]
                ob_v[ob_row, pl.ds(half + j * 16, 16)] = acc[4 + j]
            return carry2

        lax.fori_loop(0, _QB, edge_body, 0, unroll=2)

    def load_chunk(ch):
        base = sid * edges_per_tile + ch * chunk_e
        pltpu.sync_copy(src_hbm.at[pl.ds(base, chunk_e)], src_c)
        pltpu.sync_copy(dst_hbm.at[pl.ds(base, chunk_e)], dstl_c)
        pltpu.sync_copy(s_hbm.at[pl.ds(base, chunk_e)], s_c)
        for t in range(_CHUNK_B):
            src_c[pl.ds(t * _BATCH, _BATCH)] = (
                src_c[pl.ds(t * _BATCH, _BATCH)] + c * n_nodes)
            dst_c[t] = dstl_c[pl.ds(t * _BATCH, _BATCH)]

    def fire_gather(b_row, h):
        pltpu.async_copy(
            table_hbm.at[src_c.at[pl.ds(b_row * _BATCH + h * _QB, _QB)]],
            rows_v.at[h], gsems[h])

    def wait_gather(b_row, h):
        pltpu.make_async_copy(
            table_hbm.at[src_c.at[pl.ds(b_row * _BATCH + h * _QB, _QB)]],
            rows_v.at[h], gsems[h]).wait()

    def drain_scatter():
        pltpu.make_async_copy(ob_v, acc_sh.at[dst_c.at[0]], ssem).wait()

    def chunk_body(ch, carry):
        load_chunk(ch)
        fire_gather(0, 0)

        def batch_body(b, carry2):
            fire_gather(b, 1)
            wait_gather(b, 0)

            @pl.when(b > 0)
            def _():
                drain_scatter()

            compute_half(b, 0)

            @pl.when(b < _CHUNK_B - 1)
            def _():
                fire_gather(b + 1, 0)

            wait_gather(b, 1)
            compute_half(b, 1)
            pltpu.async_copy(ob_v, acc_sh.at[dst_c.at[b]], ssem, add=True)
            return carry2

        lax.fori_loop(0, _CHUNK_B, batch_body, 0, unroll=False)
        drain_scatter()
        return carry

    lax.fori_loop(0, n_chunks, chunk_body, 0, unroll=False)
    plsc.subcore_barrier()

    @pl.when(sid == 0)
    def _():
        pltpu.sync_copy(acc_sh, out_hbm.at[c])


def _sc_edge(table, src, dst, s_flat, zeros_nc, n_nodes, e_edges, k_taps, half):
    mesh = plsc.VectorSubcoreMesh(core_axis_name="c", subcore_axis_name="s")
    row_w = k_taps * half
    kern = functools.partial(
        pl.kernel,
        mesh=mesh,
        out_type=jax.ShapeDtypeStruct((2, n_nodes, 2 * half), jnp.float32),
        scratch_types=[
            pltpu.VMEM((_CHUNK_B * _BATCH,), jnp.int32),
            pltpu.VMEM((_CHUNK_B * _BATCH,), jnp.int32),
            pltpu.VMEM((_CHUNK_B, _BATCH), jnp.int32),
            pltpu.VMEM((_CHUNK_B * _BATCH, 48), jnp.float32),
            pltpu.VMEM((2, _QB, row_w), jnp.int32),
            pltpu.VMEM((_BATCH, 2 * half), jnp.float32),
            pltpu.VMEM_SHARED((n_nodes, 2 * half), jnp.float32),
            pltpu.SemaphoreType.DMA,
            pltpu.SemaphoreType.DMA,
            pltpu.SemaphoreType.DMA,
        ],
    )
    body = functools.partial(_sc_edge_body, n_nodes=n_nodes, e_edges=e_edges,
                             k_taps=k_taps, half=half)
    return kern(body)(table, src, dst, s_flat, zeros_nc)


def kernel(x, supp_edges, supp_sten, w1, off1, w2, off2, b1, b2, res_wr, res_wi):
    n, cin = x.shape
    e = supp_edges.shape[0]
    k_taps = supp_sten.shape[1] * supp_sten.shape[2]
    cout = w1.shape[2]
    half = cout // 2
    bn = 400

    xr = jnp.real(x)
    xi = jnp.imag(x)
    src = supp_edges[:, 0]
    dst = supp_edges[:, 1]
    sten = supp_sten.reshape(e, k_taps)
    s_flat = jnp.concatenate(
        [jnp.real(sten), jnp.imag(sten),
         jnp.zeros((e, 48 - 2 * k_taps), jnp.float32)], axis=1)
    zeros_nc = jnp.zeros((n, cout), jnp.float32)

    def pack_w(w):
        return (w.transpose(1, 0, 2)
                .reshape(cin, k_taps, 2, half)
                .transpose(2, 0, 1, 3)
                .reshape(2, cin, k_taps * half))

    wr1, wi1 = _prep_w(pack_w(w1), pack_w(off1))
    wr2, wi2 = _prep_w(pack_w(w2), pack_w(off2))

    table1 = _mm_v(xr, xi, wr1, wi1, bn)
    o1 = _sc_edge(table1, src, dst, s_flat, zeros_nc, n, e, k_taps, half)
    o1 = o1.reshape(2, n, cout)
    h1r = jnp.concatenate([o1[0, :, :half], o1[1, :, :half]], axis=1)
    h1i = jnp.concatenate([o1[0, :, half:], o1[1, :, half:]], axis=1)

    table2 = _mm_v(h1r, h1i, wr2, wi2, bn, fused_b=b1.reshape(1, cout))
    o2 = _sc_edge(table2, src, dst, s_flat, zeros_nc, n, e, k_taps, half)
    o2 = o2.reshape(2, n, cout)
    h2r = jnp.concatenate([o2[0, :, :half], o2[1, :, :half]], axis=1)
    h2i = jnp.concatenate([o2[0, :, half:], o2[1, :, half:]], axis=1)

    outr, outi = _final(xr, xi, res_wr, res_wi, h2r, h2i, b2.reshape(1, cout), bn)
    return lax.complex(outr, outi)
